# baseline (speedup 1.0000x reference)
import jax
import jax.numpy as jnp
from jax import lax
from jax.experimental import pallas as pl
from jax.experimental.pallas import tpu as pltpu

N, C, H, W = 1000, 81, 28, 28
HW = H * W
HB = 56


def _tc_body(lab_ref, xt_ref, o_ref):
    lab = lab_ref[...]
    cio = lax.broadcasted_iota(jnp.int32, (C, N), 0)
    m = cio == lab
    for hb in range(HB):
        slab = xt_ref[hb]
        red = jnp.sum(jnp.where(m, slab, 0.0), axis=0, keepdims=True)
        o_ref[pl.ds(hb, 1), :] = 1.0 / (1.0 + jnp.exp(-red))


@jax.jit
def _select_sigmoid(lab2, xt):
    return pl.pallas_call(
        _tc_body,
        grid=(HW // HB,),
        in_specs=[
            pl.BlockSpec((1, N), lambda i: (0, 0)),
            pl.BlockSpec((HB, C, N), lambda i: (i, 0, 0)),
        ],
        out_specs=pl.BlockSpec((HB, N), lambda i: (i, 0)),
        out_shape=jax.ShapeDtypeStruct((HW, N), jnp.float32),
        compiler_params=pltpu.CompilerParams(
            dimension_semantics=("arbitrary",),
        ),
    )(lab2, xt)


def kernel(x, labels):
    xt = x.transpose(2, 3, 1, 0).reshape(HW, C, N)
    lab2 = labels.astype(jnp.int32).reshape(1, N)
    o = _select_sigmoid(lab2, xt)
    return o.transpose(1, 0).reshape(N, 1, H, W)
